# Initial kernel scaffold; baseline (speedup 1.0000x reference)
import jax
import jax.numpy as jnp
from jax.experimental import pallas as pl


def kernel(map_designs, start_maps, goal_maps):
    raise NotImplementedError("write your pallas kernel here")


# speedup vs baseline: 11.6800x; 11.6800x over previous
import math

import jax
import jax.numpy as jnp
from jax.experimental import pallas as pl

B, H, W = 32, 32, 32
HW = H * W
G_RATIO = 0.5
TMAX = 0.25
TB_FACTOR = 0.001
T = int(TMAX * H * W)


def _heuristic(goal_maps):
    b, h, w = goal_maps.shape
    ys, xs = jnp.meshgrid(jnp.arange(h), jnp.arange(w), indexing='ij')
    loc = jnp.stack([ys, xs], axis=0).astype(jnp.float32)
    goal_idx = jnp.argmax(goal_maps.reshape(b, -1), axis=-1)
    gy = (goal_idx // w).astype(jnp.float32)
    gx = (goal_idx % w).astype(jnp.float32)
    goal_loc = jnp.stack([gy, gx], axis=1)[:, :, None, None]
    dxdy = jnp.abs(loc[None] - goal_loc)
    hmap = dxdy.sum(axis=1) - dxdy.min(axis=1)
    euc = jnp.sqrt((dxdy ** 2).sum(axis=1))
    return hmap + TB_FACTOR * euc


def _astar_body(hm_ref, cost_ref, goal_ref, start_ref, out_ref):
    hm = hm_ref[...]
    cost = cost_ref[...]
    goal = goal_ref[...]
    open0 = start_ref[...]
    obst = cost

    iota = jax.lax.broadcasted_iota(jnp.int32, (B, HW), 1)
    r_cell = iota // W
    c_cell = iota % W

    def step(_, carry):
        g, open_m, hist = carry
        f = G_RATIO * g + (1.0 - G_RATIO) * hm
        v = jnp.exp(-1.0 * f / math.sqrt(W)) * open_m
        s = jnp.maximum(jnp.sum(v, axis=-1, keepdims=True), 1e-30)
        y = v / s
        m = jnp.max(y, axis=-1, keepdims=True)
        sel_idx = jnp.min(jnp.where(y == m, iota, HW), axis=-1, keepdims=True)
        sel_mask = iota == sel_idx
        q = (1.0 - m) + m
        sel = jnp.where(sel_mask, q, 0.0)

        dist = jnp.sum(sel * goal, axis=-1, keepdims=True)
        is_unsolved = (dist < 1e-8).astype(jnp.float32)
        hist = jnp.clip(hist + sel, 0.0, 1.0)
        open_m = jnp.clip(open_m - is_unsolved * sel, 0.0, 1.0)

        rs = sel_idx // W
        cs = sel_idx % W
        window = ((jnp.abs(r_cell - rs) <= 1) & (jnp.abs(c_cell - cs) <= 1)
                  & jnp.logical_not(sel_mask))
        nb = jnp.where(window, q, 0.0) * obst
        gplus = jnp.sum((g + cost) * sel, axis=-1, keepdims=True)
        g2 = jnp.where(window, gplus, 0.0)

        idxf = (((1.0 - open_m) * (1.0 - hist) * nb
                 + open_m * (g > g2).astype(jnp.float32) * nb) > 0
                ).astype(jnp.float32)
        g = g2 * idxf + g * (1.0 - idxf)
        open_m = jnp.clip(open_m + idxf, 0.0, 1.0)
        return (g, open_m, hist)

    zeros = jnp.zeros_like(open0)
    _, _, hist = jax.lax.fori_loop(0, T, step, (zeros, open0, zeros))
    out_ref[...] = hist


def kernel(map_designs, start_maps, goal_maps):
    hm = (_heuristic(goal_maps) + map_designs).reshape(B, HW)
    cost = map_designs.reshape(B, HW)
    goal = goal_maps.reshape(B, HW)
    start = start_maps.reshape(B, HW)
    hist = pl.pallas_call(
        _astar_body,
        out_shape=jax.ShapeDtypeStruct((B, HW), jnp.float32),
    )(hm, cost, goal, start)
    return hist.reshape(B, H, W)
